# baseline (speedup 1.0000x reference)
import functools

import jax
import jax.numpy as jnp
from jax.experimental import pallas as pl
from jax.experimental.pallas import tpu as pltpu


def _mm_kernel(x_ref, w_ref, o_ref):
    o_ref[...] = jax.lax.dot(x_ref[...], w_ref[...],
                             preferred_element_type=jnp.float32)


def _matmul(x, w):
    m, k = x.shape
    k2, n = w.shape
    blk = 2000
    grid = (m // blk,)
    return pl.pallas_call(
        _mm_kernel,
        grid=grid,
        in_specs=[pl.BlockSpec((blk, k), lambda i: (i, 0)),
                  pl.BlockSpec((k, n), lambda i: (0, 0))],
        out_specs=pl.BlockSpec((blk, n), lambda i: (i, 0)),
        out_shape=jax.ShapeDtypeStruct((m, n), jnp.float32),
    )(x, w)


def _gcn_structure(row, col, edge_valid, node_valid, num_nodes):
    nonself = edge_valid & (row != col)
    deg = jax.ops.segment_sum(nonself.astype(jnp.float32), col, num_segments=num_nodes)
    deg = deg + node_valid.astype(jnp.float32)
    dinv = jnp.where(deg > 0, deg ** -0.5, 0.0).astype(jnp.float32)
    norm = (dinv[row] * dinv[col]) * nonself.astype(jnp.float32)
    loop_norm = (dinv * dinv) * node_valid.astype(jnp.float32)
    return norm, loop_norm


def _gcn_conv_from_h(h, row, col, norm, loop_norm, b, num_nodes):
    msg = h[row] * norm[:, None]
    agg = jax.ops.segment_sum(msg, col, num_segments=num_nodes)
    agg = agg + h * loop_norm[:, None]
    return agg + b


def _edge_softmax(raw, dst, edge_valid, num_nodes):
    raw = jnp.where(edge_valid, raw, -jnp.inf)
    m = jax.ops.segment_max(raw, dst, num_segments=num_nodes)
    m = jnp.where(jnp.isfinite(m), m, 0.0)
    ex = jnp.exp(raw - m[dst])
    s = jax.ops.segment_sum(ex, dst, num_segments=num_nodes)
    return ex / jnp.maximum(s[dst], 1e-16)


def _merge_structure(score, src, dst, edge_valid, node_valid, num_nodes):
    num_edges = src.shape[0]
    order = jnp.argsort(jnp.where(edge_valid, -score, jnp.inf), stable=True)
    cluster0 = jnp.full((num_nodes,), -1, dtype=jnp.int32)
    mask0 = jnp.ones((num_nodes,), dtype=bool)
    eoc0 = jnp.zeros((num_nodes,), dtype=jnp.int32)

    def body(t, st):
        cluster, mask, eoc, i = st
        eidx = order[t]
        s = src[eidx]
        d = dst[eidx]
        take = edge_valid[eidx] & mask[s] & mask[d]
        cluster = cluster.at[s].set(jnp.where(take, i, cluster[s]))
        cluster = cluster.at[d].set(jnp.where(take, i, cluster[d]))
        mask = mask.at[s].set(jnp.where(take, False, mask[s]))
        mask = mask.at[d].set(jnp.where(take, False, mask[d]))
        eoc = eoc.at[i].set(jnp.where(take, eidx, eoc[i]))
        i = i + take.astype(jnp.int32)
        return cluster, mask, eoc, i

    cluster, mask, eoc, nm = jax.lax.fori_loop(
        0, num_edges, body, (cluster0, mask0, eoc0, jnp.int32(0)))
    rem = mask & node_valid
    ranks = jnp.cumsum(rem.astype(jnp.int32)) - 1
    cluster = jnp.where(rem, nm + ranks, cluster)
    n = nm + jnp.sum(rem.astype(jnp.int32))
    cluster = jnp.where(node_valid, cluster, num_nodes)
    return cluster, n, eoc, nm


def _coalesce(src, dst, n):
    key = jnp.sort(src * n + dst)
    valid = jnp.concatenate([jnp.ones((1,), bool), key[1:] != key[:-1]])
    return key // n, key % n, valid


def _pool_apply(x, e, cluster, eoc, nm, num_nodes):
    new_x = jax.ops.segment_sum(x, cluster, num_segments=num_nodes + 1)[:num_nodes]
    j = jnp.arange(num_nodes, dtype=jnp.int32)
    scale = jnp.where(j < nm, e[eoc], 1.0)
    return new_x * scale[:, None]


def kernel(x, edge_index, batch, W1, b1, lin1_w, lin1_b, W2, b2, lin2_w, lin2_b):
    N = x.shape[0]
    B = 16
    src1, dst1 = edge_index[0], edge_index[1]
    all_edges = jnp.ones((src1.shape[0],), dtype=bool)
    all_nodes = jnp.ones((N,), dtype=bool)
    arangeN = jnp.arange(N, dtype=jnp.int32)
    norm1, loop1 = _gcn_structure(src1, dst1, all_edges, all_nodes, N)
    hmat = _matmul(x, W1)
    h = jax.nn.relu(_gcn_conv_from_h(hmat, src1, dst1, norm1, loop1, b1, N))
    raw1 = (jnp.concatenate([h[src1], h[dst1]], axis=-1) @ lin1_w + lin1_b).reshape(-1)
    e1 = _edge_softmax(raw1, dst1, all_edges, N) + 0.5
    cl1, n1, eoc1, nm1 = _merge_structure(e1, src1, dst1, all_edges, all_nodes, N)
    x1 = _pool_apply(h, e1, cl1, eoc1, nm1, N)
    src2, dst2, ev2 = _coalesce(cl1[src1], cl1[dst1], n1)
    nodes1 = arangeN < n1
    idx1 = jnp.clip(jax.ops.segment_max(arangeN, cl1, num_segments=N + 1)[:N], 0, N - 1)
    batch1 = batch[idx1]
    norm2, loop2 = _gcn_structure(src2, dst2, ev2, nodes1, N)
    h2mat = _matmul(x1, W2)
    h2 = _gcn_conv_from_h(h2mat, src2, dst2, norm2, loop2, b2, N)
    raw2 = (jnp.concatenate([h2[src2], h2[dst2]], axis=-1) @ lin2_w + lin2_b).reshape(-1)
    e2 = _edge_softmax(raw2, dst2, ev2, N) + 0.5
    cl2, n2, eoc2, nm2 = _merge_structure(e2, src2, dst2, ev2, nodes1, N)
    x2 = _pool_apply(h2, e2, cl2, eoc2, nm2, N)
    idx2 = jnp.clip(jax.ops.segment_max(arangeN, cl2, num_segments=N + 1)[:N], 0, N - 1)
    batch2 = batch1[idx2]
    clusters2 = arangeN < n2
    batch2 = jnp.where(clusters2, batch2, B)
    sums = jax.ops.segment_sum(x2, batch2, num_segments=B + 1)[:B]
    cnt = jax.ops.segment_sum(clusters2.astype(jnp.float32), batch2, num_segments=B + 1)[:B]
    cnt = jnp.maximum(cnt, 1.0)
    mean = sums / cnt[:, None]
    return jax.nn.log_softmax(mean, axis=1)


# speedup vs baseline: 18.6506x; 18.6506x over previous
import functools

import jax
import jax.numpy as jnp
from jax import lax
from jax.experimental import pallas as pl
from jax.experimental.pallas import tpu as pltpu
from jax.experimental.pallas import tpu_sc as plsc

_E = 160000
_N = 10000
_CH = 16000


def _mm_kernel(x_ref, w_ref, o_ref):
    o_ref[...] = jax.lax.dot(x_ref[...], w_ref[...],
                             preferred_element_type=jnp.float32)


def _matmul(x, w):
    m, k = x.shape
    k2, n = w.shape
    blk = 2000
    grid = (m // blk,)
    return pl.pallas_call(
        _mm_kernel,
        grid=grid,
        in_specs=[pl.BlockSpec((blk, k), lambda i: (i, 0)),
                  pl.BlockSpec((k, n), lambda i: (0, 0))],
        out_specs=pl.BlockSpec((blk, n), lambda i: (i, 0)),
        out_shape=jax.ShapeDtypeStruct((m, n), jnp.float32),
    )(x, w)


def _gcn_structure(row, col, edge_valid, node_valid, num_nodes):
    nonself = edge_valid & (row != col)
    deg = jax.ops.segment_sum(nonself.astype(jnp.float32), col, num_segments=num_nodes)
    deg = deg + node_valid.astype(jnp.float32)
    dinv = jnp.where(deg > 0, deg ** -0.5, 0.0).astype(jnp.float32)
    norm = (dinv[row] * dinv[col]) * nonself.astype(jnp.float32)
    loop_norm = (dinv * dinv) * node_valid.astype(jnp.float32)
    return norm, loop_norm


def _gcn_conv_from_h(h, row, col, norm, loop_norm, b, num_nodes):
    msg = h[row] * norm[:, None]
    agg = jax.ops.segment_sum(msg, col, num_segments=num_nodes)
    agg = agg + h * loop_norm[:, None]
    return agg + b


def _edge_softmax(raw, dst, edge_valid, num_nodes):
    raw = jnp.where(edge_valid, raw, -jnp.inf)
    m = jax.ops.segment_max(raw, dst, num_segments=num_nodes)
    m = jnp.where(jnp.isfinite(m), m, 0.0)
    ex = jnp.exp(raw - m[dst])
    s = jax.ops.segment_sum(ex, dst, num_segments=num_nodes)
    return ex / jnp.maximum(s[dst], 1e-16)


def _sload(ref, idx):
    return ref[pl.ds(idx, 16)][0]


def _sstore(ref, idx, val):
    v = ref[pl.ds(idx, 16)]
    lane = lax.iota(jnp.int32, 16)
    ref[pl.ds(idx, 16)] = jnp.where(lane == 0, val, v)


def _merge_sc_body(so_hbm, do_hbm, eo_hbm, params_hbm,
                   cluster_hbm, eoc_hbm, counts_hbm,
                   s_v, d_v, e_v, mask_v, cluster_v, eoc_v, cnt_v, sem):
    cid = lax.axis_index("c")
    sid = lax.axis_index("s")

    @pl.when((cid == 0) & (sid == 0))
    def _():
        def initbody(j, carry):
            mask_v[pl.ds(j * 16, 16)] = jnp.ones((16,), jnp.int32)
            eoc_v[pl.ds(j * 16, 16)] = jnp.zeros((16,), jnp.int32)
            return carry
        lax.fori_loop(0, (_N + 16) // 16, initbody, jnp.int32(0))

        pltpu.sync_copy(params_hbm, cnt_v)
        cvec = cnt_v[...]
        nvalid = cvec[0]
        nvn = cvec[1]

        i = jnp.int32(0)
        for c in range(_E // _CH):
            base = c * _CH
            pltpu.sync_copy(so_hbm.at[pl.ds(base, _CH)], s_v.at[pl.ds(0, _CH)])
            pltpu.sync_copy(do_hbm.at[pl.ds(base, _CH)], d_v.at[pl.ds(0, _CH)])
            pltpu.sync_copy(eo_hbm.at[pl.ds(base, _CH)], e_v.at[pl.ds(0, _CH)])
            hi = jnp.clip(nvalid - base, 0, _CH)

            def step(t, i):
                s = _sload(s_v, t)
                d = _sload(d_v, t)
                free = (_sload(mask_v, s) != 0) & (_sload(mask_v, d) != 0)

                def take(i):
                    _sstore(cluster_v, s, i)
                    _sstore(cluster_v, d, i)
                    _sstore(mask_v, s, jnp.int32(0))
                    _sstore(mask_v, d, jnp.int32(0))
                    _sstore(eoc_v, i, _sload(e_v, t))
                    return i + 1

                return lax.cond(free, take, lambda i: i, i)

            i = lax.fori_loop(0, hi, step, i)
        nm = i

        def rem_body(j, off):
            jvec = j * 16 + lax.iota(jnp.int32, 16)
            mvec = mask_v[pl.ds(j * 16, 16)]
            remv = (mvec != 0) & (jvec < nvn)
            inc = remv.astype(jnp.int32)
            csum = jnp.cumsum(inc)
            rank = off + csum - 1
            clv = cluster_v[pl.ds(j * 16, 16)]
            newcl = jnp.where(remv, nm + rank,
                              jnp.where(jvec < nvn, clv, jnp.int32(_N)))
            cluster_v[pl.ds(j * 16, 16)] = newcl
            return off + jnp.sum(inc)
        total_rem = lax.fori_loop(0, _N // 16, rem_body, jnp.int32(0))

        lane = lax.iota(jnp.int32, 16)
        cnt_v[...] = jnp.where(lane == 0, nm,
                               jnp.where(lane == 1, nm + total_rem,
                                         jnp.int32(0)))
        pltpu.sync_copy(cluster_v.at[pl.ds(0, _N)], cluster_hbm)
        pltpu.sync_copy(eoc_v.at[pl.ds(0, _N)], eoc_hbm)
        pltpu.sync_copy(cnt_v, counts_hbm)


def _merge_structure(score, src, dst, edge_valid, node_valid, num_nodes):
    num_edges = src.shape[0]
    key = jnp.where(edge_valid, -score, jnp.inf)
    eidx = jnp.arange(num_edges, dtype=jnp.int32)
    _, so, do, eo = lax.sort((key, src, dst, eidx), num_keys=1, is_stable=True)
    nvalid = jnp.sum(edge_valid.astype(jnp.int32))
    nvn = jnp.sum(node_valid.astype(jnp.int32))
    params = jnp.zeros((16,), jnp.int32).at[0].set(nvalid).at[1].set(nvn)

    mesh = plsc.VectorSubcoreMesh(core_axis_name="c", subcore_axis_name="s")
    cluster, eoc, counts = pl.kernel(
        _merge_sc_body,
        mesh=mesh,
        compiler_params=pltpu.CompilerParams(needs_layout_passes=False),
        out_type=[jax.ShapeDtypeStruct((num_nodes,), jnp.int32),
                  jax.ShapeDtypeStruct((num_nodes,), jnp.int32),
                  jax.ShapeDtypeStruct((16,), jnp.int32)],
        scratch_types=[pltpu.VMEM((_CH + 16,), jnp.int32),
                       pltpu.VMEM((_CH + 16,), jnp.int32),
                       pltpu.VMEM((_CH + 16,), jnp.int32),
                       pltpu.VMEM((num_nodes + 16,), jnp.int32),
                       pltpu.VMEM((num_nodes + 16,), jnp.int32),
                       pltpu.VMEM((num_nodes + 16,), jnp.int32),
                       pltpu.VMEM((16,), jnp.int32),
                       pltpu.SemaphoreType.DMA],
    )(so, do, eo, params)
    nm = counts[0]
    n = counts[1]
    return cluster, n, eoc, nm


def _coalesce(src, dst, n):
    key = jnp.sort(src * n + dst)
    valid = jnp.concatenate([jnp.ones((1,), bool), key[1:] != key[:-1]])
    return key // n, key % n, valid


def _pool_apply(x, e, cluster, eoc, nm, num_nodes):
    new_x = jax.ops.segment_sum(x, cluster, num_segments=num_nodes + 1)[:num_nodes]
    j = jnp.arange(num_nodes, dtype=jnp.int32)
    scale = jnp.where(j < nm, e[eoc], 1.0)
    return new_x * scale[:, None]


def kernel(x, edge_index, batch, W1, b1, lin1_w, lin1_b, W2, b2, lin2_w, lin2_b):
    N = x.shape[0]
    B = 16
    src1, dst1 = edge_index[0], edge_index[1]
    all_edges = jnp.ones((src1.shape[0],), dtype=bool)
    all_nodes = jnp.ones((N,), dtype=bool)
    arangeN = jnp.arange(N, dtype=jnp.int32)
    norm1, loop1 = _gcn_structure(src1, dst1, all_edges, all_nodes, N)
    hmat = _matmul(x, W1)
    h = jax.nn.relu(_gcn_conv_from_h(hmat, src1, dst1, norm1, loop1, b1, N))
    raw1 = (jnp.concatenate([h[src1], h[dst1]], axis=-1) @ lin1_w + lin1_b).reshape(-1)
    e1 = _edge_softmax(raw1, dst1, all_edges, N) + 0.5
    cl1, n1, eoc1, nm1 = _merge_structure(e1, src1, dst1, all_edges, all_nodes, N)
    x1 = _pool_apply(h, e1, cl1, eoc1, nm1, N)
    src2, dst2, ev2 = _coalesce(cl1[src1], cl1[dst1], n1)
    nodes1 = arangeN < n1
    idx1 = jnp.clip(jax.ops.segment_max(arangeN, cl1, num_segments=N + 1)[:N], 0, N - 1)
    batch1 = batch[idx1]
    norm2, loop2 = _gcn_structure(src2, dst2, ev2, nodes1, N)
    h2mat = _matmul(x1, W2)
    h2 = _gcn_conv_from_h(h2mat, src2, dst2, norm2, loop2, b2, N)
    raw2 = (jnp.concatenate([h2[src2], h2[dst2]], axis=-1) @ lin2_w + lin2_b).reshape(-1)
    e2 = _edge_softmax(raw2, dst2, ev2, N) + 0.5
    cl2, n2, eoc2, nm2 = _merge_structure(e2, src2, dst2, ev2, nodes1, N)
    x2 = _pool_apply(h2, e2, cl2, eoc2, nm2, N)
    idx2 = jnp.clip(jax.ops.segment_max(arangeN, cl2, num_segments=N + 1)[:N], 0, N - 1)
    batch2 = batch1[idx2]
    clusters2 = arangeN < n2
    batch2 = jnp.where(clusters2, batch2, B)
    sums = jax.ops.segment_sum(x2, batch2, num_segments=B + 1)[:B]
    cnt = jax.ops.segment_sum(clusters2.astype(jnp.float32), batch2, num_segments=B + 1)[:B]
    cnt = jnp.maximum(cnt, 1.0)
    mean = sums / cnt[:, None]
    return jax.nn.log_softmax(mean, axis=1)


# speedup vs baseline: 19.8338x; 1.0634x over previous
import functools

import jax
import jax.numpy as jnp
from jax import lax
from jax.experimental import pallas as pl
from jax.experimental.pallas import tpu as pltpu
from jax.experimental.pallas import tpu_sc as plsc

_E = 160000
_N = 10000
_CH = 16000


def _mm_kernel(x_ref, w_ref, o_ref):
    o_ref[...] = jax.lax.dot(x_ref[...], w_ref[...],
                             preferred_element_type=jnp.float32)


def _matmul(x, w):
    m, k = x.shape
    k2, n = w.shape
    blk = 2000
    grid = (m // blk,)
    return pl.pallas_call(
        _mm_kernel,
        grid=grid,
        in_specs=[pl.BlockSpec((blk, k), lambda i: (i, 0)),
                  pl.BlockSpec((k, n), lambda i: (0, 0))],
        out_specs=pl.BlockSpec((blk, n), lambda i: (i, 0)),
        out_shape=jax.ShapeDtypeStruct((m, n), jnp.float32),
    )(x, w)


def _gcn_structure(row, col, edge_valid, node_valid, num_nodes):
    nonself = edge_valid & (row != col)
    deg = jax.ops.segment_sum(nonself.astype(jnp.float32), col, num_segments=num_nodes)
    deg = deg + node_valid.astype(jnp.float32)
    dinv = jnp.where(deg > 0, deg ** -0.5, 0.0).astype(jnp.float32)
    norm = (dinv[row] * dinv[col]) * nonself.astype(jnp.float32)
    loop_norm = (dinv * dinv) * node_valid.astype(jnp.float32)
    return norm, loop_norm


def _gcn_conv_from_h(h, row, col, norm, loop_norm, b, num_nodes):
    msg = h[row] * norm[:, None]
    agg = jax.ops.segment_sum(msg, col, num_segments=num_nodes)
    agg = agg + h * loop_norm[:, None]
    return agg + b


def _edge_softmax(raw, dst, edge_valid, num_nodes):
    raw = jnp.where(edge_valid, raw, -jnp.inf)
    m = jnp.max(raw)
    m = jnp.where(jnp.isfinite(m), m, 0.0)
    ex = jnp.exp(raw - m)
    s = jax.ops.segment_sum(ex, dst, num_segments=num_nodes)
    return ex / jnp.maximum(s[dst], 1e-16)


def _sload(ref, idx):
    return ref[pl.ds(idx, 16)][0]


def _sstore(ref, idx, val):
    v = ref[pl.ds(idx, 16)]
    lane = lax.iota(jnp.int32, 16)
    ref[pl.ds(idx, 16)] = jnp.where(lane == 0, val, v)


def _merge_sc_body(so_hbm, do_hbm, eo_hbm, params_hbm,
                   cluster_hbm, eoc_hbm, icl_hbm, counts_hbm,
                   s_v, d_v, e_v, mask_v, cluster_v, eoc_v, icl_v, cnt_v, sem):
    cid = lax.axis_index("c")
    sid = lax.axis_index("s")

    @pl.when((cid == 0) & (sid == 0))
    def _():
        def initbody(j, carry):
            mask_v[pl.ds(j * 16, 16)] = jnp.ones((16,), jnp.int32)
            eoc_v[pl.ds(j * 16, 16)] = jnp.zeros((16,), jnp.int32)
            icl_v[pl.ds(j * 16, 16)] = jnp.zeros((16,), jnp.int32)
            return carry
        lax.fori_loop(0, (_N + 16) // 16, initbody, jnp.int32(0))

        pltpu.sync_copy(params_hbm, cnt_v)
        cvec = cnt_v[...]
        nvalid = cvec[0]
        nvn = cvec[1]

        i = jnp.int32(0)
        for c in range(_E // _CH):
            base = c * _CH
            pltpu.sync_copy(so_hbm.at[pl.ds(base, _CH)], s_v.at[pl.ds(0, _CH)])
            pltpu.sync_copy(do_hbm.at[pl.ds(base, _CH)], d_v.at[pl.ds(0, _CH)])
            pltpu.sync_copy(eo_hbm.at[pl.ds(base, _CH)], e_v.at[pl.ds(0, _CH)])
            hi = jnp.clip(nvalid - base, 0, _CH)

            def step(t, i):
                s = _sload(s_v, t)
                d = _sload(d_v, t)
                free = (_sload(mask_v, s) != 0) & (_sload(mask_v, d) != 0)

                def take(i):
                    _sstore(cluster_v, s, i)
                    _sstore(cluster_v, d, i)
                    _sstore(mask_v, s, jnp.int32(0))
                    _sstore(mask_v, d, jnp.int32(0))
                    _sstore(eoc_v, i, _sload(e_v, t))
                    _sstore(icl_v, i, jnp.maximum(s, d))
                    return i + 1

                return lax.cond(free, take, lambda i: i, i)

            i = lax.fori_loop(0, hi, step, i)
        nm = i

        def rem_body(j, off):
            jvec = j * 16 + lax.iota(jnp.int32, 16)
            mvec = mask_v[pl.ds(j * 16, 16)]
            remv = (mvec != 0) & (jvec < nvn)
            inc = remv.astype(jnp.int32)
            csum = jnp.cumsum(inc)
            rank = off + csum - 1
            clv = cluster_v[pl.ds(j * 16, 16)]
            newcl = jnp.where(remv, nm + rank,
                              jnp.where(jvec < nvn, clv, jnp.int32(_N)))
            cluster_v[pl.ds(j * 16, 16)] = newcl
            plsc.store_scatter(icl_v, [nm + rank], jvec, mask=remv)
            return off + jnp.sum(inc)
        total_rem = lax.fori_loop(0, _N // 16, rem_body, jnp.int32(0))

        lane = lax.iota(jnp.int32, 16)
        cnt_v[...] = jnp.where(lane == 0, nm,
                               jnp.where(lane == 1, nm + total_rem,
                                         jnp.int32(0)))
        pltpu.sync_copy(cluster_v.at[pl.ds(0, _N)], cluster_hbm)
        pltpu.sync_copy(eoc_v.at[pl.ds(0, _N)], eoc_hbm)
        pltpu.sync_copy(icl_v.at[pl.ds(0, _N)], icl_hbm)
        pltpu.sync_copy(cnt_v, counts_hbm)


def _merge_structure(score, src, dst, edge_valid, node_valid, num_nodes):
    num_edges = src.shape[0]
    key = jnp.where(edge_valid, -score, jnp.inf)
    eidx = jnp.arange(num_edges, dtype=jnp.int32)
    _, so, do, eo = lax.sort((key, src, dst, eidx), num_keys=1, is_stable=True)
    nvalid = jnp.sum(edge_valid.astype(jnp.int32))
    nvn = jnp.sum(node_valid.astype(jnp.int32))
    params = jnp.zeros((16,), jnp.int32).at[0].set(nvalid).at[1].set(nvn)

    mesh = plsc.VectorSubcoreMesh(core_axis_name="c", subcore_axis_name="s")
    cluster, eoc, icl, counts = pl.kernel(
        _merge_sc_body,
        mesh=mesh,
        compiler_params=pltpu.CompilerParams(needs_layout_passes=False),
        out_type=[jax.ShapeDtypeStruct((num_nodes,), jnp.int32),
                  jax.ShapeDtypeStruct((num_nodes,), jnp.int32),
                  jax.ShapeDtypeStruct((num_nodes,), jnp.int32),
                  jax.ShapeDtypeStruct((16,), jnp.int32)],
        scratch_types=[pltpu.VMEM((_CH + 16,), jnp.int32),
                       pltpu.VMEM((_CH + 16,), jnp.int32),
                       pltpu.VMEM((_CH + 16,), jnp.int32),
                       pltpu.VMEM((num_nodes + 16,), jnp.int32),
                       pltpu.VMEM((num_nodes + 16,), jnp.int32),
                       pltpu.VMEM((num_nodes + 16,), jnp.int32),
                       pltpu.VMEM((num_nodes + 16,), jnp.int32),
                       pltpu.VMEM((16,), jnp.int32),
                       pltpu.SemaphoreType.DMA],
    )(so, do, eo, params)
    nm = counts[0]
    n = counts[1]
    return cluster, n, eoc, nm, icl


def _coalesce(src, dst, n):
    key = jnp.sort(src * n + dst)
    valid = jnp.concatenate([jnp.ones((1,), bool), key[1:] != key[:-1]])
    return key // n, key % n, valid


def _pool_apply(x, e, cluster, eoc, nm, num_nodes):
    new_x = jax.ops.segment_sum(x, cluster, num_segments=num_nodes + 1)[:num_nodes]
    j = jnp.arange(num_nodes, dtype=jnp.int32)
    scale = jnp.where(j < nm, e[eoc], 1.0)
    return new_x * scale[:, None]


def kernel(x, edge_index, batch, W1, b1, lin1_w, lin1_b, W2, b2, lin2_w, lin2_b):
    N = x.shape[0]
    B = 16
    src1, dst1 = edge_index[0], edge_index[1]
    all_edges = jnp.ones((src1.shape[0],), dtype=bool)
    all_nodes = jnp.ones((N,), dtype=bool)
    arangeN = jnp.arange(N, dtype=jnp.int32)
    norm1, loop1 = _gcn_structure(src1, dst1, all_edges, all_nodes, N)
    hmat = _matmul(x, W1)
    h = jax.nn.relu(_gcn_conv_from_h(hmat, src1, dst1, norm1, loop1, b1, N))
    raw1 = (jnp.concatenate([h[src1], h[dst1]], axis=-1) @ lin1_w + lin1_b).reshape(-1)
    e1 = _edge_softmax(raw1, dst1, all_edges, N) + 0.5
    cl1, n1, eoc1, nm1, icl1 = _merge_structure(e1, src1, dst1, all_edges, all_nodes, N)
    x1 = _pool_apply(h, e1, cl1, eoc1, nm1, N)
    src2, dst2, ev2 = _coalesce(cl1[src1], cl1[dst1], n1)
    nodes1 = arangeN < n1
    batch1 = batch[icl1]
    norm2, loop2 = _gcn_structure(src2, dst2, ev2, nodes1, N)
    h2mat = _matmul(x1, W2)
    h2 = _gcn_conv_from_h(h2mat, src2, dst2, norm2, loop2, b2, N)
    raw2 = (jnp.concatenate([h2[src2], h2[dst2]], axis=-1) @ lin2_w + lin2_b).reshape(-1)
    e2 = _edge_softmax(raw2, dst2, ev2, N) + 0.5
    cl2, n2, eoc2, nm2, icl2 = _merge_structure(e2, src2, dst2, ev2, nodes1, N)
    x2 = _pool_apply(h2, e2, cl2, eoc2, nm2, N)
    batch2 = batch1[icl2]
    clusters2 = arangeN < n2
    batch2 = jnp.where(clusters2, batch2, B)
    sums = jax.ops.segment_sum(x2, batch2, num_segments=B + 1)[:B]
    cnt = jax.ops.segment_sum(clusters2.astype(jnp.float32), batch2, num_segments=B + 1)[:B]
    cnt = jnp.maximum(cnt, 1.0)
    mean = sums / cnt[:, None]
    return jax.nn.log_softmax(mean, axis=1)


# speedup vs baseline: 35.9601x; 1.8131x over previous
import functools

import jax
import jax.numpy as jnp
from jax import lax
from jax.experimental import pallas as pl
from jax.experimental.pallas import tpu as pltpu
from jax.experimental.pallas import tpu_sc as plsc

_E = 160000
_N = 10000
_CH = 16000


def _mm_kernel(x_ref, w_ref, o_ref):
    o_ref[...] = jax.lax.dot(x_ref[...], w_ref[...],
                             preferred_element_type=jnp.float32)


def _matmul(x, w):
    m, k = x.shape
    k2, n = w.shape
    blk = 2000
    grid = (m // blk,)
    return pl.pallas_call(
        _mm_kernel,
        grid=grid,
        in_specs=[pl.BlockSpec((blk, k), lambda i: (i, 0)),
                  pl.BlockSpec((k, n), lambda i: (0, 0))],
        out_specs=pl.BlockSpec((blk, n), lambda i: (i, 0)),
        out_shape=jax.ShapeDtypeStruct((m, n), jnp.float32),
    )(x, w)


def _gcn_structure(row, col, edge_valid, node_valid, num_nodes):
    nonself = edge_valid & (row != col)
    deg = jax.ops.segment_sum(nonself.astype(jnp.float32), col, num_segments=num_nodes)
    deg = deg + node_valid.astype(jnp.float32)
    dinv = jnp.where(deg > 0, deg ** -0.5, 0.0).astype(jnp.float32)
    norm = (dinv[row] * dinv[col]) * nonself.astype(jnp.float32)
    loop_norm = (dinv * dinv) * node_valid.astype(jnp.float32)
    return norm, loop_norm


def _gcn_conv_from_h(h, row, col, norm, loop_norm, b, num_nodes):
    msg = h[row] * norm[:, None]
    agg = jax.ops.segment_sum(msg, col, num_segments=num_nodes)
    agg = agg + h * loop_norm[:, None]
    return agg + b


def _edge_softmax(raw, dst, edge_valid, num_nodes):
    raw = jnp.where(edge_valid, raw, -jnp.inf)
    m = jnp.max(raw)
    m = jnp.where(jnp.isfinite(m), m, 0.0)
    ex = jnp.exp(raw - m)
    s = jax.ops.segment_sum(ex, dst, num_segments=num_nodes)
    return ex / jnp.maximum(s[dst], 1e-16)


def _sload(ref, idx):
    return ref[pl.ds(idx, 16)][0]


def _sstore(ref, idx, val):
    v = ref[pl.ds(idx, 16)]
    lane = lax.iota(jnp.int32, 16)
    ref[pl.ds(idx, 16)] = jnp.where(lane == 0, val, v)


def _merge_sc_body(so_hbm, do_hbm, eo_hbm, params_hbm,
                   cluster_hbm, eoc_hbm, icl_hbm, counts_hbm,
                   s_v, d_v, e_v, cluster_v, eoc_v, icl_v, cnt_v, sem):
    cid = lax.axis_index("c")
    sid = lax.axis_index("s")

    @pl.when((cid == 0) & (sid == 0))
    def _():
        lane = lax.iota(jnp.int32, 16)

        def initbody(j, carry):
            cluster_v[pl.ds(j * 16, 16)] = jnp.full((16,), -1, jnp.int32)
            eoc_v[pl.ds(j * 16, 16)] = jnp.zeros((16,), jnp.int32)
            icl_v[pl.ds(j * 16, 16)] = jnp.zeros((16,), jnp.int32)
            return carry
        lax.fori_loop(0, (_N + 16) // 16, initbody, jnp.int32(0))

        pltpu.sync_copy(params_hbm, cnt_v)
        cvec = cnt_v[...]
        nvalid = cvec[0]
        nvn = cvec[1]

        def chunk(c, i):
            base = c * _CH
            pltpu.sync_copy(so_hbm.at[pl.ds(base, _CH)], s_v.at[pl.ds(0, _CH)])
            pltpu.sync_copy(do_hbm.at[pl.ds(base, _CH)], d_v.at[pl.ds(0, _CH)])
            pltpu.sync_copy(eo_hbm.at[pl.ds(base, _CH)], e_v.at[pl.ds(0, _CH)])
            hi = jnp.clip(nvalid - base, 0, _CH)
            ghi = (hi + 15) // 16

            def group(g, i):
                s_vec = s_v[pl.ds(g * 16, 16)]
                d_vec = d_v[pl.ds(g * 16, 16)]
                e_vec = e_v[pl.ds(g * 16, 16)]
                cs = plsc.load_gather(cluster_v, [s_vec])
                cd = plsc.load_gather(cluster_v, [d_vec])
                tvec = g * 16 + lane
                cand = (cs < 0) & (cd < 0) & (tvec < hi)
                candi = jnp.where(cand, jnp.int32(1), jnp.int32(0))
                ncand = plsc.all_reduce_population_count(cand)

                def hasc(i):
                    for k in range(16):
                        def dolane(i, k=k):
                            s = s_vec[k]
                            d = d_vec[k]
                            free = ((_sload(cluster_v, s) < 0)
                                    & (_sload(cluster_v, d) < 0))

                            def take(i):
                                _sstore(cluster_v, s, i)
                                _sstore(cluster_v, d, i)
                                _sstore(eoc_v, i, e_vec[k])
                                _sstore(icl_v, i, jnp.maximum(s, d))
                                return i + 1

                            return lax.cond(free, take, lambda i: i, i)
                        i = lax.cond(candi[k] != 0, dolane, lambda i: i, i)
                    return i

                return lax.cond(ncand[0] > 0, hasc, lambda i: i, i)

            return lax.fori_loop(0, ghi, group, i)

        nm = lax.fori_loop(0, _E // _CH, chunk, jnp.int32(0))

        def rem_body(j, off):
            jvec = j * 16 + lane
            clv = cluster_v[pl.ds(j * 16, 16)]
            remv = (clv < 0) & (jvec < nvn)
            inc = jnp.where(remv, jnp.int32(1), jnp.int32(0))
            csum = jnp.cumsum(inc)
            rank = off + csum - 1
            newcl = jnp.where(remv, nm + rank,
                              jnp.where(jvec < nvn, clv, jnp.int32(_N)))
            cluster_v[pl.ds(j * 16, 16)] = newcl
            plsc.store_scatter(icl_v, [nm + rank], jvec, mask=remv)
            return off + jnp.sum(inc)
        total_rem = lax.fori_loop(0, _N // 16, rem_body, jnp.int32(0))

        lane = lax.iota(jnp.int32, 16)
        cnt_v[...] = jnp.where(lane == 0, nm,
                               jnp.where(lane == 1, nm + total_rem,
                                         jnp.int32(0)))
        pltpu.sync_copy(cluster_v.at[pl.ds(0, _N)], cluster_hbm)
        pltpu.sync_copy(eoc_v.at[pl.ds(0, _N)], eoc_hbm)
        pltpu.sync_copy(icl_v.at[pl.ds(0, _N)], icl_hbm)
        pltpu.sync_copy(cnt_v, counts_hbm)


def _merge_structure(score, src, dst, edge_valid, node_valid, num_nodes):
    num_edges = src.shape[0]
    key = jnp.where(edge_valid, -score, jnp.inf)
    eidx = jnp.arange(num_edges, dtype=jnp.int32)
    _, so, do, eo = lax.sort((key, src, dst, eidx), num_keys=1, is_stable=True)
    nvalid = jnp.sum(edge_valid.astype(jnp.int32))
    nvn = jnp.sum(node_valid.astype(jnp.int32))
    params = jnp.zeros((16,), jnp.int32).at[0].set(nvalid).at[1].set(nvn)

    mesh = plsc.VectorSubcoreMesh(core_axis_name="c", subcore_axis_name="s")
    cluster, eoc, icl, counts = pl.kernel(
        _merge_sc_body,
        mesh=mesh,
        compiler_params=pltpu.CompilerParams(needs_layout_passes=False),
        out_type=[jax.ShapeDtypeStruct((num_nodes,), jnp.int32),
                  jax.ShapeDtypeStruct((num_nodes,), jnp.int32),
                  jax.ShapeDtypeStruct((num_nodes,), jnp.int32),
                  jax.ShapeDtypeStruct((16,), jnp.int32)],
        scratch_types=[pltpu.VMEM((_CH + 16,), jnp.int32),
                       pltpu.VMEM((_CH + 16,), jnp.int32),
                       pltpu.VMEM((_CH + 16,), jnp.int32),
                       pltpu.VMEM((num_nodes + 16,), jnp.int32),
                       pltpu.VMEM((num_nodes + 16,), jnp.int32),
                       pltpu.VMEM((num_nodes + 16,), jnp.int32),
                       pltpu.VMEM((16,), jnp.int32),
                       pltpu.SemaphoreType.DMA],
    )(so, do, eo, params)
    nm = counts[0]
    n = counts[1]
    return cluster, n, eoc, nm, icl


def _coalesce(src, dst, n):
    key = jnp.sort(src * n + dst)
    valid = jnp.concatenate([jnp.ones((1,), bool), key[1:] != key[:-1]])
    return key // n, key % n, valid


def _pool_apply(x, e, cluster, eoc, nm, num_nodes):
    new_x = jax.ops.segment_sum(x, cluster, num_segments=num_nodes + 1)[:num_nodes]
    j = jnp.arange(num_nodes, dtype=jnp.int32)
    scale = jnp.where(j < nm, e[eoc], 1.0)
    return new_x * scale[:, None]


def kernel(x, edge_index, batch, W1, b1, lin1_w, lin1_b, W2, b2, lin2_w, lin2_b):
    N = x.shape[0]
    B = 16
    src1, dst1 = edge_index[0], edge_index[1]
    all_edges = jnp.ones((src1.shape[0],), dtype=bool)
    all_nodes = jnp.ones((N,), dtype=bool)
    arangeN = jnp.arange(N, dtype=jnp.int32)
    norm1, loop1 = _gcn_structure(src1, dst1, all_edges, all_nodes, N)
    hmat = _matmul(x, W1)
    h = jax.nn.relu(_gcn_conv_from_h(hmat, src1, dst1, norm1, loop1, b1, N))
    raw1 = (jnp.concatenate([h[src1], h[dst1]], axis=-1) @ lin1_w + lin1_b).reshape(-1)
    e1 = _edge_softmax(raw1, dst1, all_edges, N) + 0.5
    cl1, n1, eoc1, nm1, icl1 = _merge_structure(e1, src1, dst1, all_edges, all_nodes, N)
    x1 = _pool_apply(h, e1, cl1, eoc1, nm1, N)
    src2, dst2, ev2 = _coalesce(cl1[src1], cl1[dst1], n1)
    nodes1 = arangeN < n1
    batch1 = batch[icl1]
    norm2, loop2 = _gcn_structure(src2, dst2, ev2, nodes1, N)
    h2mat = _matmul(x1, W2)
    h2 = _gcn_conv_from_h(h2mat, src2, dst2, norm2, loop2, b2, N)
    raw2 = (jnp.concatenate([h2[src2], h2[dst2]], axis=-1) @ lin2_w + lin2_b).reshape(-1)
    e2 = _edge_softmax(raw2, dst2, ev2, N) + 0.5
    cl2, n2, eoc2, nm2, icl2 = _merge_structure(e2, src2, dst2, ev2, nodes1, N)
    x2 = _pool_apply(h2, e2, cl2, eoc2, nm2, N)
    batch2 = batch1[icl2]
    clusters2 = arangeN < n2
    batch2 = jnp.where(clusters2, batch2, B)
    sums = jax.ops.segment_sum(x2, batch2, num_segments=B + 1)[:B]
    cnt = jax.ops.segment_sum(clusters2.astype(jnp.float32), batch2, num_segments=B + 1)[:B]
    cnt = jnp.maximum(cnt, 1.0)
    mean = sums / cnt[:, None]
    return jax.nn.log_softmax(mean, axis=1)
